# baseline (speedup 1.0000x reference)
import jax
import jax.numpy as jnp
from jax.experimental import pallas as pl
from jax.experimental.pallas import tpu as pltpu

N_EMBED = 1024
EMBED_DIM = 64
BETA = 0.25

N_TOKENS = 4 * 8 * 32 * 32
N_SPATIAL = 8 * 32 * 32
BLK_T = 4096
BLK_PER_B = N_SPATIAL // BLK_T
NUM_BLK = N_TOKENS // BLK_T


def _vq_body(zc_ref, emb_ref, embt_ref,
             enc_ref, zqc_ref, idx_ref, loss_ref, ppl_ref,
             esq_ref, counts_ref, lacc_ref):
    i = pl.program_id(0)
    emb = emb_ref[...]
    embt = embt_ref[...]

    @pl.when(i == 0)
    def _init():
        esq_ref[...] = jnp.sum(embt * embt, axis=0)[None, :]
        counts_ref[...] = jnp.zeros_like(counts_ref)
        lacc_ref[...] = jnp.zeros_like(lacc_ref)
        loss_ref[...] = jnp.zeros_like(loss_ref)
        ppl_ref[...] = jnp.zeros_like(ppl_ref)

    zb = zc_ref[0].T
    zsq = jnp.sum(zb * zb, axis=1, keepdims=True)
    mm = jnp.dot(zb, embt, preferred_element_type=jnp.float32)
    d = (zsq + esq_ref[...]) - 2.0 * mm

    dmin = jnp.min(d, axis=1, keepdims=True)
    iota = jax.lax.broadcasted_iota(jnp.int32, (BLK_T, N_EMBED), 1)
    idx = jnp.min(jnp.where(d == dmin, iota, N_EMBED), axis=1)
    enc = (iota == idx[:, None]).astype(jnp.float32)
    enc_ref[...] = enc

    zq = jax.lax.dot_general(
        enc, emb, (((1,), (0,)), ((), ())),
        preferred_element_type=jnp.float32)
    zqc_ref[0] = (zb + (zq - zb)).T
    idx_ref[...] = idx.reshape(1, 1, BLK_T)

    counts_ref[...] += jnp.sum(enc, axis=0)[None, :]
    lacc_ref[...] += jnp.sum((zq - zb) ** 2)[None, None]

    @pl.when(i == NUM_BLK - 1)
    def _fini():
        p = counts_ref[...] * (1.0 / N_TOKENS)
        ent = jnp.sum(p * jnp.log(p + 1e-10))
        ppl_ref[...] = jnp.exp(-ent)[None, None]
        loss_ref[...] = lacc_ref[...] * (BETA / (N_TOKENS * EMBED_DIM))


def _vq_call(z3, emb, embt):
    return pl.pallas_call(
        _vq_body,
        grid=(NUM_BLK,),
        in_specs=[
            pl.BlockSpec((1, EMBED_DIM, BLK_T),
                         lambda i: (i // BLK_PER_B, 0, i % BLK_PER_B)),
            pl.BlockSpec((N_EMBED, EMBED_DIM), lambda i: (0, 0)),
            pl.BlockSpec((EMBED_DIM, N_EMBED), lambda i: (0, 0)),
        ],
        out_specs=[
            pl.BlockSpec((BLK_T, N_EMBED), lambda i: (i, 0)),
            pl.BlockSpec((1, EMBED_DIM, BLK_T),
                         lambda i: (i // BLK_PER_B, 0, i % BLK_PER_B)),
            pl.BlockSpec((1, 1, BLK_T), lambda i: (i, 0, 0)),
            pl.BlockSpec((1, 1), lambda i: (0, 0)),
            pl.BlockSpec((1, 1), lambda i: (0, 0)),
        ],
        out_shape=[
            jax.ShapeDtypeStruct((N_TOKENS, N_EMBED), jnp.float32),
            jax.ShapeDtypeStruct((4, EMBED_DIM, N_SPATIAL), jnp.float32),
            jax.ShapeDtypeStruct((NUM_BLK, 1, BLK_T), jnp.int32),
            jax.ShapeDtypeStruct((1, 1), jnp.float32),
            jax.ShapeDtypeStruct((1, 1), jnp.float32),
        ],
        scratch_shapes=[
            pltpu.VMEM((1, N_EMBED), jnp.float32),
            pltpu.VMEM((1, N_EMBED), jnp.float32),
            pltpu.VMEM((1, 1), jnp.float32),
        ],
    )(z3, emb, embt)


def kernel(z, embedding_weight):
    b, c, dd, h, w = z.shape
    z3 = z.reshape(b, c, dd * h * w)
    embt = embedding_weight.T
    enc, zqc, idx3, loss2, ppl2 = _vq_call(z3, embedding_weight, embt)
    z_q_out = zqc.reshape(b, c, dd, h, w)
    encoding_indices = idx3.reshape(N_TOKENS)
    return (z_q_out, loss2[0, 0], ppl2[0, 0], enc, encoding_indices)


# speedup vs baseline: 1.3083x; 1.3083x over previous
import jax
import jax.numpy as jnp
from jax.experimental import pallas as pl
from jax.experimental.pallas import tpu as pltpu

N_EMBED = 1024
EMBED_DIM = 64
BETA = 0.25

N_TOKENS = 4 * 8 * 32 * 32
BLK_T = 4096
NUM_BLK = N_TOKENS // BLK_T


def _vq_body(zb_ref, emb_ref, embt_ref,
             enc_ref, zq_ref, idx_ref, loss_ref, ppl_ref,
             esq_ref, counts_ref, lacc_ref):
    i = pl.program_id(0)
    emb = emb_ref[...]
    embt = embt_ref[...]

    @pl.when(i == 0)
    def _init():
        esq_ref[...] = jnp.sum(embt * embt, axis=0)[None, :]
        counts_ref[...] = jnp.zeros_like(counts_ref)
        lacc_ref[...] = jnp.zeros_like(lacc_ref)
        loss_ref[...] = jnp.zeros_like(loss_ref)
        ppl_ref[...] = jnp.zeros_like(ppl_ref)

    zb = zb_ref[...]
    zsq = jnp.sum(zb * zb, axis=1, keepdims=True)
    mm = jnp.dot(zb, embt, preferred_element_type=jnp.float32)
    d = (zsq + esq_ref[...]) - 2.0 * mm

    dmin = jnp.min(d, axis=1, keepdims=True)
    iota = jax.lax.broadcasted_iota(jnp.int32, (BLK_T, N_EMBED), 1)
    idx = jnp.min(jnp.where(d == dmin, iota, N_EMBED), axis=1)
    enc = (iota == idx[:, None]).astype(jnp.float32)
    enc_ref[...] = enc

    zq = jax.lax.dot_general(
        enc, emb, (((1,), (0,)), ((), ())),
        preferred_element_type=jnp.float32)
    zq_ref[...] = zb + (zq - zb)
    idx_ref[...] = idx.reshape(1, 1, BLK_T)

    counts_ref[...] += jnp.sum(enc, axis=0)[None, :]
    lacc_ref[...] += jnp.sum((zq - zb) ** 2)[None, None]

    @pl.when(i == NUM_BLK - 1)
    def _fini():
        p = counts_ref[...] * (1.0 / N_TOKENS)
        ent = jnp.sum(p * jnp.log(p + 1e-10))
        ppl_ref[...] = jnp.exp(-ent)[None, None]
        loss_ref[...] = lacc_ref[...] * (BETA / (N_TOKENS * EMBED_DIM))


def _vq_call(z_flat, emb, embt):
    return pl.pallas_call(
        _vq_body,
        grid=(NUM_BLK,),
        in_specs=[
            pl.BlockSpec((BLK_T, EMBED_DIM), lambda i: (i, 0)),
            pl.BlockSpec((N_EMBED, EMBED_DIM), lambda i: (0, 0)),
            pl.BlockSpec((EMBED_DIM, N_EMBED), lambda i: (0, 0)),
        ],
        out_specs=[
            pl.BlockSpec((BLK_T, N_EMBED), lambda i: (i, 0)),
            pl.BlockSpec((BLK_T, EMBED_DIM), lambda i: (i, 0)),
            pl.BlockSpec((1, 1, BLK_T), lambda i: (i, 0, 0)),
            pl.BlockSpec((1, 1), lambda i: (0, 0)),
            pl.BlockSpec((1, 1), lambda i: (0, 0)),
        ],
        out_shape=[
            jax.ShapeDtypeStruct((N_TOKENS, N_EMBED), jnp.float32),
            jax.ShapeDtypeStruct((N_TOKENS, EMBED_DIM), jnp.float32),
            jax.ShapeDtypeStruct((NUM_BLK, 1, BLK_T), jnp.int32),
            jax.ShapeDtypeStruct((1, 1), jnp.float32),
            jax.ShapeDtypeStruct((1, 1), jnp.float32),
        ],
        scratch_shapes=[
            pltpu.VMEM((1, N_EMBED), jnp.float32),
            pltpu.VMEM((1, N_EMBED), jnp.float32),
            pltpu.VMEM((1, 1), jnp.float32),
        ],
    )(z_flat, emb, embt)


def kernel(z, embedding_weight):
    b, c, dd, h, w = z.shape
    zp = jnp.transpose(z, (0, 2, 3, 4, 1))
    z_flat = zp.reshape(-1, c)
    embt = embedding_weight.T
    enc, zq_st, idx3, loss2, ppl2 = _vq_call(z_flat, embedding_weight, embt)
    z_q_out = jnp.transpose(zq_st.reshape(b, dd, h, w, c), (0, 4, 1, 2, 3))
    encoding_indices = idx3.reshape(N_TOKENS)
    return (z_q_out, loss2[0, 0], ppl2[0, 0], enc, encoding_indices)


# speedup vs baseline: 1.3569x; 1.0372x over previous
import jax
import jax.numpy as jnp
from jax.experimental import pallas as pl
from jax.experimental.pallas import tpu as pltpu

N_EMBED = 1024
EMBED_DIM = 64
BETA = 0.25

N_TOKENS = 4 * 8 * 32 * 32
BLK_T = 2048
NUM_BLK = N_TOKENS // BLK_T


def _vq_body(zb_ref, emb_ref, embt2_ref,
             enc_ref, zq_ref, idx_ref, loss_ref, ppl_ref,
             esq_ref, counts_ref, lacc_ref):
    i = pl.program_id(0)
    emb = emb_ref[...]
    embt2 = embt2_ref[...]

    @pl.when(i == 0)
    def _init():
        esq_ref[...] = jnp.sum((0.5 * embt2) * (0.5 * embt2), axis=0)[None, :]
        counts_ref[...] = jnp.zeros_like(counts_ref)
        lacc_ref[...] = jnp.zeros_like(lacc_ref)
        loss_ref[...] = jnp.zeros_like(loss_ref)
        ppl_ref[...] = jnp.zeros_like(ppl_ref)

    zb = zb_ref[...]
    zsq = jnp.sum(zb * zb, axis=1, keepdims=True)
    mm2 = jnp.dot(zb, embt2, preferred_element_type=jnp.float32)
    d = (zsq + esq_ref[...]) - mm2

    dmin = jnp.min(d, axis=1, keepdims=True)
    iotaf = jax.lax.broadcasted_iota(
        jnp.int32, (1, N_EMBED), 1).astype(jnp.float32)
    idxf = jnp.min(jnp.where(d == dmin, iotaf, float(N_EMBED)), axis=1)
    enc = (iotaf == idxf[:, None]).astype(jnp.float32)
    enc_ref[...] = enc
    idx = idxf.astype(jnp.int32)

    zq = jax.lax.dot_general(
        enc, emb, (((1,), (0,)), ((), ())),
        preferred_element_type=jnp.float32)
    zq_ref[...] = zb + (zq - zb)
    idx_ref[...] = idx[:, None]

    counts_ref[...] += jnp.sum(enc, axis=0)[None, :]
    lacc_ref[...] += jnp.sum((zq - zb) ** 2)[None, None]

    @pl.when(i == NUM_BLK - 1)
    def _fini():
        p = counts_ref[...] * (1.0 / N_TOKENS)
        ent = jnp.sum(p * jnp.log(p + 1e-10))
        ppl_ref[...] = jnp.exp(-ent)[None, None]
        loss_ref[...] = lacc_ref[...] * (BETA / (N_TOKENS * EMBED_DIM))


def _vq_call(z_flat, emb, embt2):
    return pl.pallas_call(
        _vq_body,
        grid=(NUM_BLK,),
        in_specs=[
            pl.BlockSpec((BLK_T, EMBED_DIM), lambda i: (i, 0)),
            pl.BlockSpec((N_EMBED, EMBED_DIM), lambda i: (0, 0)),
            pl.BlockSpec((EMBED_DIM, N_EMBED), lambda i: (0, 0)),
        ],
        out_specs=[
            pl.BlockSpec((BLK_T, N_EMBED), lambda i: (i, 0)),
            pl.BlockSpec((BLK_T, EMBED_DIM), lambda i: (i, 0)),
            pl.BlockSpec((BLK_T, 1), lambda i: (i, 0)),
            pl.BlockSpec((1, 1), lambda i: (0, 0)),
            pl.BlockSpec((1, 1), lambda i: (0, 0)),
        ],
        out_shape=[
            jax.ShapeDtypeStruct((N_TOKENS, N_EMBED), jnp.float32),
            jax.ShapeDtypeStruct((N_TOKENS, EMBED_DIM), jnp.float32),
            jax.ShapeDtypeStruct((N_TOKENS, 1), jnp.int32),
            jax.ShapeDtypeStruct((1, 1), jnp.float32),
            jax.ShapeDtypeStruct((1, 1), jnp.float32),
        ],
        scratch_shapes=[
            pltpu.VMEM((1, N_EMBED), jnp.float32),
            pltpu.VMEM((1, N_EMBED), jnp.float32),
            pltpu.VMEM((1, 1), jnp.float32),
        ],
    )(z_flat, emb, embt2)


def kernel(z, embedding_weight):
    b, c, dd, h, w = z.shape
    zp = jnp.transpose(z, (0, 2, 3, 4, 1))
    z_flat = zp.reshape(-1, c)
    embt2 = embedding_weight.T * 2.0
    enc, zq_st, idx2, loss2, ppl2 = _vq_call(z_flat, embedding_weight, embt2)
    z_q_out = jnp.transpose(zq_st.reshape(b, dd, h, w, c), (0, 4, 1, 2, 3))
    encoding_indices = idx2.reshape(N_TOKENS)
    return (z_q_out, loss2[0, 0], ppl2[0, 0], enc, encoding_indices)
